# Initial kernel scaffold; baseline (speedup 1.0000x reference)
import jax
import jax.numpy as jnp
from jax.experimental import pallas as pl


def kernel(class_labels, bboxes, scores, sides, states, cat_table, side_table, state_table, W_box, b_box, W_score, b_score, gamma, beta):
    raise NotImplementedError("write your pallas kernel here")


# speedup vs baseline: 3.5650x; 3.5650x over previous
import jax
import jax.numpy as jnp
from jax.experimental import pallas as pl

B, L, H = 1024, 50, 768
T = B * L
EPS = 0.1
N_BLK = 1024


def _onehot_nopad(idx, k, n):
    cols = jax.lax.broadcasted_iota(jnp.int32, (n, k), 1)
    return jnp.where((idx[:, None] == cols) & (cols > 0), 1.0, 0.0).astype(jnp.float32)


def _body(cls_ref, sid_ref, sta_ref, box_ref, sco_ref,
          cat_ref, side_ref, state_ref, wboxT_ref, wscT_ref,
          bbox_ref, bsc_ref, gamma_ref, beta_ref, out_ref):
    n = out_ref.shape[0]
    c = cls_ref[:, 0]
    s = sid_ref[:, 0]
    t = sta_ref[:, 0]

    emb = jnp.dot(_onehot_nopad(c, 3, n), cat_ref[...],
                  preferred_element_type=jnp.float32)
    emb += jnp.dot(_onehot_nopad(s, 3, n), side_ref[...],
                   preferred_element_type=jnp.float32)
    emb += jnp.dot(_onehot_nopad(t, 6, n), state_ref[...],
                   preferred_element_type=jnp.float32)
    emb += jnp.dot(box_ref[...], wboxT_ref[...],
                   preferred_element_type=jnp.float32)
    emb += sco_ref[...] * wscT_ref[...]
    emb += bbox_ref[...] + bsc_ref[...]

    mu = jnp.mean(emb, axis=-1, keepdims=True)
    d = emb - mu
    var = jnp.mean(d * d, axis=-1, keepdims=True)
    out_ref[...] = d * jax.lax.rsqrt(var + EPS) * gamma_ref[...] + beta_ref[...]


def kernel(class_labels, bboxes, scores, sides, states, cat_table, side_table,
           state_table, W_box, b_box, W_score, b_score, gamma, beta):
    cls = class_labels.reshape(T, 1)
    sid = sides.reshape(T, 1)
    sta = states.reshape(T, 1)
    box = bboxes.reshape(T, 4)
    sco = scores.reshape(T, 1)
    wboxT = W_box.T
    wscT = W_score.T
    bbox2 = b_box.reshape(1, H)
    bsc2 = b_score.reshape(1, H)
    gamma2 = gamma.reshape(1, H)
    beta2 = beta.reshape(1, H)

    grid = (T // N_BLK,)

    def tok(i):
        return (i, 0)

    def rep(i):
        return (0, 0)

    out = pl.pallas_call(
        _body,
        grid=grid,
        in_specs=[
            pl.BlockSpec((N_BLK, 1), tok),
            pl.BlockSpec((N_BLK, 1), tok),
            pl.BlockSpec((N_BLK, 1), tok),
            pl.BlockSpec((N_BLK, 4), tok),
            pl.BlockSpec((N_BLK, 1), tok),
            pl.BlockSpec((3, H), rep),
            pl.BlockSpec((3, H), rep),
            pl.BlockSpec((6, H), rep),
            pl.BlockSpec((4, H), rep),
            pl.BlockSpec((1, H), rep),
            pl.BlockSpec((1, H), rep),
            pl.BlockSpec((1, H), rep),
            pl.BlockSpec((1, H), rep),
            pl.BlockSpec((1, H), rep),
        ],
        out_specs=pl.BlockSpec((N_BLK, H), tok),
        out_shape=jax.ShapeDtypeStruct((T, H), jnp.float32),
    )(cls, sid, sta, box, sco, cat_table, side_table, state_table,
      wboxT, wscT, bbox2, bsc2, gamma2, beta2)

    return out.reshape(B, L, H)


# speedup vs baseline: 3.5879x; 1.0064x over previous
import jax
import jax.numpy as jnp
from jax.experimental import pallas as pl

B, L, H = 1024, 50, 768
T = B * L
EPS = 0.1
N_BLK = 2048


def _onehot_nopad(idx, k, n):
    cols = jax.lax.broadcasted_iota(jnp.int32, (n, k), 1)
    return jnp.where((idx[:, None] == cols) & (cols > 0), 1.0, 0.0).astype(jnp.float32)


def _body(cls_ref, sid_ref, sta_ref, box_ref, sco_ref,
          cat_ref, side_ref, state_ref, wboxT_ref, wscT_ref,
          bbox_ref, bsc_ref, gamma_ref, beta_ref, out_ref):
    n = out_ref.shape[0]
    c = cls_ref[:, 0]
    s = sid_ref[:, 0]
    t = sta_ref[:, 0]

    emb = jnp.dot(_onehot_nopad(c, 3, n), cat_ref[...],
                  preferred_element_type=jnp.float32)
    emb += jnp.dot(_onehot_nopad(s, 3, n), side_ref[...],
                   preferred_element_type=jnp.float32)
    emb += jnp.dot(_onehot_nopad(t, 6, n), state_ref[...],
                   preferred_element_type=jnp.float32)
    emb += jnp.dot(box_ref[...], wboxT_ref[...],
                   preferred_element_type=jnp.float32)
    emb += sco_ref[...] * wscT_ref[...]
    emb += bbox_ref[...] + bsc_ref[...]

    mu = jnp.mean(emb, axis=-1, keepdims=True)
    d = emb - mu
    var = jnp.mean(d * d, axis=-1, keepdims=True)
    out_ref[...] = d * jax.lax.rsqrt(var + EPS) * gamma_ref[...] + beta_ref[...]


def kernel(class_labels, bboxes, scores, sides, states, cat_table, side_table,
           state_table, W_box, b_box, W_score, b_score, gamma, beta):
    cls = class_labels.reshape(T, 1)
    sid = sides.reshape(T, 1)
    sta = states.reshape(T, 1)
    box = bboxes.reshape(T, 4)
    sco = scores.reshape(T, 1)
    wboxT = W_box.T
    wscT = W_score.T
    bbox2 = b_box.reshape(1, H)
    bsc2 = b_score.reshape(1, H)
    gamma2 = gamma.reshape(1, H)
    beta2 = beta.reshape(1, H)

    grid = (T // N_BLK,)

    def tok(i):
        return (i, 0)

    def rep(i):
        return (0, 0)

    out = pl.pallas_call(
        _body,
        grid=grid,
        in_specs=[
            pl.BlockSpec((N_BLK, 1), tok),
            pl.BlockSpec((N_BLK, 1), tok),
            pl.BlockSpec((N_BLK, 1), tok),
            pl.BlockSpec((N_BLK, 4), tok),
            pl.BlockSpec((N_BLK, 1), tok),
            pl.BlockSpec((3, H), rep),
            pl.BlockSpec((3, H), rep),
            pl.BlockSpec((6, H), rep),
            pl.BlockSpec((4, H), rep),
            pl.BlockSpec((1, H), rep),
            pl.BlockSpec((1, H), rep),
            pl.BlockSpec((1, H), rep),
            pl.BlockSpec((1, H), rep),
            pl.BlockSpec((1, H), rep),
        ],
        out_specs=pl.BlockSpec((N_BLK, H), tok),
        out_shape=jax.ShapeDtypeStruct((T, H), jnp.float32),
    )(cls, sid, sta, box, sco, cat_table, side_table, state_table,
      wboxT, wscT, bbox2, bsc2, gamma2, beta2)

    return out.reshape(B, L, H)


# speedup vs baseline: 4.2473x; 1.1838x over previous
import jax
import jax.numpy as jnp
from jax.experimental import pallas as pl

B, L, H = 1024, 50, 768
EPS = 0.1
N_B = 32

_DN = (((2,), (0,)), ((), ()))


def _onehot_nopad(idx, k):
    cols = jax.lax.broadcasted_iota(jnp.int32, idx.shape + (k,), 2)
    return jnp.where((idx[:, :, None] == cols) & (cols > 0), 1.0, 0.0)


def _body(cls_ref, sid_ref, sta_ref, box_ref, sco_ref,
          cat_ref, side_ref, state_ref, wboxT_ref, wscT_ref,
          bias_ref, gamma_ref, beta_ref, out_ref):
    emb = jax.lax.dot_general(_onehot_nopad(cls_ref[...], 3), cat_ref[...],
                              _DN, preferred_element_type=jnp.float32)
    emb += jax.lax.dot_general(_onehot_nopad(sid_ref[...], 3), side_ref[...],
                               _DN, preferred_element_type=jnp.float32)
    emb += jax.lax.dot_general(_onehot_nopad(sta_ref[...], 6), state_ref[...],
                               _DN, preferred_element_type=jnp.float32)
    emb += jax.lax.dot_general(box_ref[...], wboxT_ref[...],
                               _DN, preferred_element_type=jnp.float32)
    emb += sco_ref[...][:, :, None] * wscT_ref[...]
    emb += bias_ref[...]

    mu = jnp.mean(emb, axis=-1, keepdims=True)
    d = emb - mu
    var = jnp.mean(d * d, axis=-1, keepdims=True)
    out_ref[...] = d * jax.lax.rsqrt(var + EPS) * gamma_ref[...] + beta_ref[...]


def kernel(class_labels, bboxes, scores, sides, states, cat_table, side_table,
           state_table, W_box, b_box, W_score, b_score, gamma, beta):
    wboxT = W_box.T
    wscT = W_score.T.reshape(1, 1, H)
    bias = (b_box + b_score).reshape(1, 1, H)
    gamma3 = gamma.reshape(1, 1, H)
    beta3 = beta.reshape(1, 1, H)

    grid = (B // N_B,)

    def tok2(i):
        return (i, 0)

    def tok3(i):
        return (i, 0, 0)

    def rep2(i):
        return (0, 0)

    def rep3(i):
        return (0, 0, 0)

    return pl.pallas_call(
        _body,
        grid=grid,
        in_specs=[
            pl.BlockSpec((N_B, L), tok2),
            pl.BlockSpec((N_B, L), tok2),
            pl.BlockSpec((N_B, L), tok2),
            pl.BlockSpec((N_B, L, 4), tok3),
            pl.BlockSpec((N_B, L), tok2),
            pl.BlockSpec((3, H), rep2),
            pl.BlockSpec((3, H), rep2),
            pl.BlockSpec((6, H), rep2),
            pl.BlockSpec((4, H), rep2),
            pl.BlockSpec((1, 1, H), rep3),
            pl.BlockSpec((1, 1, H), rep3),
            pl.BlockSpec((1, 1, H), rep3),
            pl.BlockSpec((1, 1, H), rep3),
        ],
        out_specs=pl.BlockSpec((N_B, L, H), tok3),
        out_shape=jax.ShapeDtypeStruct((B, L, H), jnp.float32),
    )(class_labels, sides, states, bboxes, scores, cat_table, side_table,
      state_table, wboxT, wscT, bias, gamma3, beta3)


# speedup vs baseline: 5.8446x; 1.3761x over previous
import jax
import jax.numpy as jnp
from jax.experimental import pallas as pl

B, L, H = 1024, 50, 768
EPS = 0.1
N_B = 32


def _body(cls_ref, sid_ref, sta_ref, box_ref, sco_ref, w18_ref,
          gamma_ref, beta_ref, out_ref):
    c = cls_ref[...][:, :, None]
    s = sid_ref[...][:, :, None]
    t = sta_ref[...][:, :, None]
    cols = jax.lax.broadcasted_iota(jnp.int32, (N_B, L, 12), 2)
    oh = (((cols == c) & (cols >= 1))
          | ((cols == s + 3) & (cols >= 4))
          | ((cols == t + 6) & (cols >= 7)))
    feat = jnp.concatenate(
        [oh.astype(jnp.float32), box_ref[...], sco_ref[...][:, :, None],
         jnp.ones((N_B, L, 1), jnp.float32)], axis=-1)
    w18 = w18_ref[...]
    gamma = gamma_ref[...]
    beta = beta_ref[...]
    for b in range(N_B):
        emb = jnp.dot(feat[b], w18, preferred_element_type=jnp.float32)
        mu = jnp.mean(emb, axis=-1, keepdims=True)
        d = emb - mu
        var = jnp.mean(d * d, axis=-1, keepdims=True)
        out_ref[b] = d * jax.lax.rsqrt(var + EPS) * gamma + beta


def kernel(class_labels, bboxes, scores, sides, states, cat_table, side_table,
           state_table, W_box, b_box, W_score, b_score, gamma, beta):
    w18 = jnp.concatenate(
        [cat_table, side_table, state_table, W_box.T, W_score.T,
         (b_box + b_score).reshape(1, H)], axis=0)
    gamma2 = gamma.reshape(1, H)
    beta2 = beta.reshape(1, H)

    grid = (B // N_B,)

    def tok2(i):
        return (i, 0)

    def tok3(i):
        return (i, 0, 0)

    def rep2(i):
        return (0, 0)

    return pl.pallas_call(
        _body,
        grid=grid,
        in_specs=[
            pl.BlockSpec((N_B, L), tok2),
            pl.BlockSpec((N_B, L), tok2),
            pl.BlockSpec((N_B, L), tok2),
            pl.BlockSpec((N_B, L, 4), tok3),
            pl.BlockSpec((N_B, L), tok2),
            pl.BlockSpec((18, H), rep2),
            pl.BlockSpec((1, H), rep2),
            pl.BlockSpec((1, H), rep2),
        ],
        out_specs=pl.BlockSpec((N_B, L, H), tok3),
        out_shape=jax.ShapeDtypeStruct((B, L, H), jnp.float32),
    )(class_labels, sides, states, bboxes, scores, w18, gamma2, beta2)


# speedup vs baseline: 6.9356x; 1.1867x over previous
import jax
import jax.numpy as jnp
from jax.experimental import pallas as pl

B, L, H = 1024, 50, 768
EPS = 0.1
N_B = 32


def _body(cls_ref, sid_ref, sta_ref, box_ref, sco_ref, w18_ref, gext_ref,
          gamma_ref, beta_ref, out_ref):
    c = cls_ref[...][:, :, None]
    s = sid_ref[...][:, :, None]
    t = sta_ref[...][:, :, None]
    cols = jax.lax.broadcasted_iota(jnp.int32, (N_B, L, 12), 2)
    oh = (((cols == c) & (cols >= 1))
          | ((cols == s + 3) & (cols >= 4))
          | ((cols == t + 6) & (cols >= 7)))
    feat = jnp.concatenate(
        [oh.astype(jnp.float32), box_ref[...], sco_ref[...][:, :, None],
         jnp.ones((N_B, L, 1), jnp.float32)], axis=-1)
    w18 = w18_ref[...]
    gext = gext_ref[...]
    gamma = gamma_ref[...]
    beta = beta_ref[...]
    for b in range(N_B):
        f = feat[b]
        emb = jnp.dot(f, w18, preferred_element_type=jnp.float32)
        q = jnp.dot(f, gext, preferred_element_type=jnp.float32)
        mu = q[:, 18:19]
        s2 = jnp.sum(q[:, :18] * f, axis=-1, keepdims=True)
        var = s2 - mu * mu
        out_ref[b] = (emb - mu) * jax.lax.rsqrt(var + EPS) * gamma + beta


def kernel(class_labels, bboxes, scores, sides, states, cat_table, side_table,
           state_table, W_box, b_box, W_score, b_score, gamma, beta):
    w18 = jnp.concatenate(
        [cat_table, side_table, state_table, W_box.T, W_score.T,
         (b_box + b_score).reshape(1, H)], axis=0)
    gext = jnp.concatenate(
        [w18 @ w18.T / H, jnp.mean(w18, axis=1, keepdims=True)],
        axis=1)
    gamma2 = gamma.reshape(1, H)
    beta2 = beta.reshape(1, H)

    grid = (B // N_B,)

    def tok2(i):
        return (i, 0)

    def tok3(i):
        return (i, 0, 0)

    def rep2(i):
        return (0, 0)

    return pl.pallas_call(
        _body,
        grid=grid,
        in_specs=[
            pl.BlockSpec((N_B, L), tok2),
            pl.BlockSpec((N_B, L), tok2),
            pl.BlockSpec((N_B, L), tok2),
            pl.BlockSpec((N_B, L, 4), tok3),
            pl.BlockSpec((N_B, L), tok2),
            pl.BlockSpec((18, H), rep2),
            pl.BlockSpec((18, 19), rep2),
            pl.BlockSpec((1, H), rep2),
            pl.BlockSpec((1, H), rep2),
        ],
        out_specs=pl.BlockSpec((N_B, L, H), tok3),
        out_shape=jax.ShapeDtypeStruct((B, L, H), jnp.float32),
    )(class_labels, sides, states, bboxes, scores, w18, gext, gamma2, beta2)
